# baseline (device time: 173329 ns/iter reference)
import os

import jax
import jax.numpy as jnp
from jax import lax
from jax.experimental import pallas as pl
from jax.experimental.pallas import tpu as pltpu

_SKIP_COMM = bool(os.environ.get("SKIP_COMM"))

N_DEV = 4
SQ = 2048
SKV = 2048
H_PER = 8
DH = 128
DM = 1024
QC = 4
CH = SQ // N_DEV
HCH = CH // 2
SCALE = 0.08838834764831843
BLK = 64
NB = SKV // BLK
MESH = pl.DeviceIdType.MESH

_FREE = {
    0: [kb for kb in range(NB) if kb % 3 == 0],
    1: [0] + [kb for kb in range(NB) if kb % 3 == 2],
    2: [0] + [kb for kb in range(NB) if kb % 3 == 1],
}
_DIAG = {
    0: [],
    1: [kb for kb in range(NB) if kb % 3 == 1],
    2: [kb for kb in range(NB) if kb % 3 == 2],
}
_KVL = _FREE[0] + _DIAG[0] + _FREE[1] + _DIAG[1] + _FREE[2] + _DIAG[2]
_SEC = {}
_off = 0
for _r in range(3):
    _fw, _dw = len(_FREE[_r]) * BLK, len(_DIAG[_r]) * BLK
    _SEC[_r] = (_off, _fw, _dw)
    _off += _fw + _dw
KVG = _off


def kernel(x, Wq, K_ext, V_ext, Wo):
    my = lax.axis_index("i")
    xb = x.astype(jnp.bfloat16)
    Wqb = Wq.astype(jnp.bfloat16)
    Wob = Wo.astype(jnp.bfloat16)
    kvl = jnp.asarray(_KVL)

    def prep(a):
        a = lax.dynamic_slice_in_dim(a, my * H_PER, H_PER, axis=2)[0]
        a = a.transpose(1, 0, 2).astype(jnp.bfloat16)
        a = a.reshape(H_PER, NB, BLK, DH)[:, kvl]
        return a.reshape(H_PER, KVG, DH)

    Kg = prep(K_ext)
    Vg = prep(V_ext)

    def body(x_ref, wq_ref, kg_ref, vg_ref, wo_ref, out_ref,
             m1_ref, m2_ref, sendbuf, rs_recv, agb, send_sems, recv_sems):
        qc = pl.program_id(0)
        my_pos = lax.axis_index("i")
        left = lax.rem(my_pos - 1 + N_DEV, N_DEV)
        right = lax.rem(my_pos + 1, N_DEV)
        chunk = lax.rem(my_pos - qc + N_DEV, N_DEV)
        rows = pl.ds(chunk * CH, CH)

        @pl.when((qc == 0) & (not _SKIP_COMM))
        def _entry_barrier():
            barrier = pltpu.get_barrier_semaphore()
            for nbr in (left, right):
                pl.semaphore_signal(barrier, inc=1, device_id=(nbr,),
                                    device_id_type=MESH)
            pl.semaphore_wait(barrier, 2)

        @pl.when(qc == 0)
        def _make_masks():
            for mref in (m1_ref, m2_ref):
                n = mref.shape[0]
                i = lax.broadcasted_iota(jnp.int32, (n, n), 0) // BLK
                j = lax.broadcasted_iota(jnp.int32, (n, n), 1) // BLK
                mref[...] = (i == j).astype(jnp.bfloat16)

        def class_compute(c, r):
            qbs = [qb for qb in range(8 * c, 8 * c + 8) if qb % 3 == r]
            nqb = len(qbs)
            nq = nqb * BLK
            sec0, fw, dw = _SEC[r]
            j0 = (qbs[0] - r) // 3
            xq = jnp.concatenate(
                [x_ref[0, qb * BLK:(qb + 1) * BLK, :] for qb in qbs], axis=0)
            mref = m1_ref if r == 1 else m2_ref

            def hbody(h, acc):
                wqh = wq_ref[:, pl.ds(h * DH, DH)]
                qh = jnp.dot(xq, wqh, preferred_element_type=jnp.float32)
                qh = (qh * SCALE).astype(jnp.bfloat16)
                kg = kg_ref[h, sec0:sec0 + fw + dw, :]
                sc = lax.dot_general(qh, kg, (((1,), (1,)), ((), ())),
                                     preferred_element_type=jnp.float32)
                w = jnp.exp(sc)
                if dw:
                    wf32 = w[:, :fw]
                    wd32 = w[:, fw:] * mref[j0 * BLK:j0 * BLK + nq, :]
                    wf = wf32.astype(jnp.bfloat16)
                    wd = wd32.astype(jnp.bfloat16)
                    vf = vg_ref[h, sec0:sec0 + fw, :]
                    vd = vg_ref[h, sec0 + fw:sec0 + fw + dw, :]
                    denom = (jnp.sum(wf32, axis=1, keepdims=True)
                             + jnp.sum(wd32, axis=1, keepdims=True))
                    ctx = (jnp.dot(wf, vf, preferred_element_type=jnp.float32)
                           + jnp.dot(wd, vd, preferred_element_type=jnp.float32))
                else:
                    wf = w.astype(jnp.bfloat16)
                    vf = vg_ref[h, sec0:sec0 + fw, :]
                    denom = jnp.sum(w, axis=1, keepdims=True)
                    ctx = jnp.dot(wf, vf, preferred_element_type=jnp.float32)
                ctx = ctx / denom
                woh = wo_ref[pl.ds(h * DH, DH), :]
                return acc + jnp.dot(ctx.astype(jnp.bfloat16), woh,
                                     preferred_element_type=jnp.float32)

            acc = lax.fori_loop(0, H_PER, hbody,
                                jnp.zeros((nq, DM), jnp.float32))
            for i, qb in enumerate(qbs):
                out_ref[qb * BLK:(qb + 1) * BLK, :] = \
                    acc[i * BLK:(i + 1) * BLK, :]

        for c in range(N_DEV):
            @pl.when(chunk == c)
            def _compute(c=c):
                for r in range(3):
                    class_compute(c, r)

        @pl.when((qc > 0) & (not _SKIP_COMM))
        def _rs_recv_add():
            st = qc - 1
            rdma = pltpu.make_async_remote_copy(
                src_ref=sendbuf.at[st],
                dst_ref=rs_recv.at[st],
                send_sem=send_sems.at[st],
                recv_sem=recv_sems.at[st],
                device_id=(left,),
                device_id_type=MESH,
            )
            rdma.wait_recv()
            out_ref[rows, :] += rs_recv[st].astype(jnp.float32)

        @pl.when((qc < QC - 1) & (not _SKIP_COMM))
        def _rs_send():
            sendbuf[qc] = out_ref[rows, :].astype(jnp.bfloat16)
            rdma = pltpu.make_async_remote_copy(
                src_ref=sendbuf.at[qc],
                dst_ref=rs_recv.at[qc],
                send_sem=send_sems.at[qc],
                recv_sem=recv_sems.at[qc],
                device_id=(right,),
                device_id_type=MESH,
            )
            rdma.start()

        @pl.when((qc == QC - 1) & (not _SKIP_COMM))
        def _finish():
            red = lax.rem(my_pos + 1, N_DEV)
            red_rows = pl.ds(red * CH, CH)
            agb[red_rows, :] = out_ref[red_rows, :].astype(jnp.bfloat16)
            for t in range(N_DEV - 1):
                cw_c = lax.rem(my_pos + 1 - t + N_DEV, N_DEV)
                ccw_c = lax.rem(my_pos + 1 + t, N_DEV)
                cw = pltpu.make_async_remote_copy(
                    src_ref=agb.at[pl.ds(cw_c * CH, HCH), :],
                    dst_ref=agb.at[pl.ds(cw_c * CH, HCH), :],
                    send_sem=send_sems.at[3 + t],
                    recv_sem=recv_sems.at[3 + t],
                    device_id=(right,),
                    device_id_type=MESH,
                )
                ccw = pltpu.make_async_remote_copy(
                    src_ref=agb.at[pl.ds(ccw_c * CH + HCH, HCH), :],
                    dst_ref=agb.at[pl.ds(ccw_c * CH + HCH, HCH), :],
                    send_sem=send_sems.at[6 + t],
                    recv_sem=recv_sems.at[6 + t],
                    device_id=(left,),
                    device_id_type=MESH,
                )
                cw.start()
                ccw.start()
                cw.wait()
                ccw.wait()
                got_cw = pl.ds(lax.rem(my_pos - t + N_DEV, N_DEV) * CH, HCH)
                got_ccw = pl.ds(lax.rem(my_pos + 2 + t, N_DEV) * CH + HCH, HCH)
                out_ref[got_cw, :] = agb[got_cw, :].astype(jnp.float32)
                out_ref[got_ccw, :] = agb[got_ccw, :].astype(jnp.float32)
            for st in range(N_DEV - 1):
                pltpu.make_async_remote_copy(
                    src_ref=sendbuf.at[st],
                    dst_ref=rs_recv.at[st],
                    send_sem=send_sems.at[st],
                    recv_sem=recv_sems.at[st],
                    device_id=(right,),
                    device_id_type=MESH,
                ).wait_send()

    out = pl.pallas_call(
        body,
        grid=(QC,),
        in_specs=[
            pl.BlockSpec((1, SQ, DM), lambda qc: (0, 0, 0)),
            pl.BlockSpec((DM, DM), lambda qc: (0, 0)),
            pl.BlockSpec((H_PER, KVG, DH), lambda qc: (0, 0, 0)),
            pl.BlockSpec((H_PER, KVG, DH), lambda qc: (0, 0, 0)),
            pl.BlockSpec((DM, DM), lambda qc: (0, 0)),
        ],
        out_specs=pl.BlockSpec((SQ, DM), lambda qc: (0, 0)),
        out_shape=jax.ShapeDtypeStruct((SQ, DM), jnp.float32),
        scratch_shapes=[
            pltpu.VMEM((len(_DIAG[1]) * BLK,) * 2, jnp.bfloat16),
            pltpu.VMEM((len(_DIAG[2]) * BLK,) * 2, jnp.bfloat16),
            pltpu.VMEM((N_DEV - 1, CH, DM), jnp.bfloat16),
            pltpu.VMEM((N_DEV - 1, CH, DM), jnp.bfloat16),
            pltpu.VMEM((SQ, DM), jnp.bfloat16),
            pltpu.SemaphoreType.DMA((9,)),
            pltpu.SemaphoreType.DMA((9,)),
        ],
        compiler_params=pltpu.CompilerParams(collective_id=0),
    )(xb, Wqb, Kg, Vg, Wob)
    return out.reshape(1, SQ, DM)


# device time: 150510 ns/iter; 1.1516x vs baseline; 1.1516x over previous
import os

import jax
import jax.numpy as jnp
from jax import lax
from jax.experimental import pallas as pl
from jax.experimental.pallas import tpu as pltpu

_SKIP_COMM = bool(os.environ.get("SKIP_COMM"))

N_DEV = 4
SQ = 2048
SKV = 2048
H_PER = 8
DH = 128
DM = 1024
QC = 4
CH = SQ // N_DEV
HCH = CH // 2
SCALE = 0.08838834764831843
BLK = 64
MESH = pl.DeviceIdType.MESH


def kernel(x, Wq, K_ext, V_ext, Wo):
    my = lax.axis_index("i")
    xb = x.astype(jnp.bfloat16)
    Wqb = Wq.astype(jnp.bfloat16)
    Wob = Wo.astype(jnp.bfloat16)
    Kb = lax.dynamic_slice_in_dim(K_ext, my * H_PER, H_PER, axis=2)[0]
    Kb = Kb.transpose(1, 0, 2).astype(jnp.bfloat16)
    Vb = lax.dynamic_slice_in_dim(V_ext, my * H_PER, H_PER, axis=2)[0]
    Vb = Vb.transpose(1, 0, 2).astype(jnp.bfloat16)

    def body(x_ref, wq_ref, k_ref, v_ref, wo_ref, out_ref,
             bias_ref, sendbuf, rs_recv, agb, send_sems, recv_sems):
        qc = pl.program_id(0)
        my_pos = lax.axis_index("i")
        left = lax.rem(my_pos - 1 + N_DEV, N_DEV)
        right = lax.rem(my_pos + 1, N_DEV)
        chunk = lax.rem(my_pos - qc + N_DEV, N_DEV)
        rows = pl.ds(chunk * CH, CH)

        @pl.when((qc == 0) & (not _SKIP_COMM))
        def _entry_barrier():
            barrier = pltpu.get_barrier_semaphore()
            for nbr in (left, right):
                pl.semaphore_signal(barrier, inc=1, device_id=(nbr,),
                                    device_id_type=MESH)
            pl.semaphore_wait(barrier, 2)

        rowb = (lax.broadcasted_iota(jnp.int32, (CH, SKV), 0) + chunk * CH) // BLK
        colb = lax.broadcasted_iota(jnp.int32, (CH, SKV), 1) // BLK
        keep = (rowb == colb) | (colb == 0) | (lax.rem(rowb + colb, 3) == 0)
        bias_ref[...] = jnp.where(keep, 0.0, -1e9).astype(jnp.float32)

        xc = x_ref[0, rows, :]
        acc = jnp.zeros((CH, DM), jnp.float32)
        for h in range(H_PER):
            kh = k_ref[h]
            vh = v_ref[h]
            qh = jnp.dot(xc, wq_ref[:, h * DH:(h + 1) * DH],
                         preferred_element_type=jnp.float32)
            qh = (qh * SCALE).astype(jnp.bfloat16)
            sc = lax.dot_general(qh, kh, (((1,), (1,)), ((), ())),
                                 preferred_element_type=jnp.float32)
            w = jnp.exp((sc + bias_ref[...]).astype(jnp.bfloat16))
            denom = jnp.sum(w, axis=1, keepdims=True, dtype=jnp.float32)
            ctx = jnp.dot(w, vh, preferred_element_type=jnp.float32) / denom
            acc = acc + jnp.dot(ctx.astype(jnp.bfloat16),
                                wo_ref[h * DH:(h + 1) * DH, :],
                                preferred_element_type=jnp.float32)
        out_ref[rows, :] = acc

        @pl.when((qc > 0) & (not _SKIP_COMM))
        def _rs_recv_add():
            st = qc - 1
            rdma = pltpu.make_async_remote_copy(
                src_ref=sendbuf.at[st],
                dst_ref=rs_recv.at[st],
                send_sem=send_sems.at[st],
                recv_sem=recv_sems.at[st],
                device_id=(left,),
                device_id_type=MESH,
            )
            rdma.wait_recv()
            out_ref[rows, :] += rs_recv[st].astype(jnp.float32)

        @pl.when((qc < QC - 1) & (not _SKIP_COMM))
        def _rs_send():
            sendbuf[qc] = out_ref[rows, :].astype(jnp.bfloat16)
            rdma = pltpu.make_async_remote_copy(
                src_ref=sendbuf.at[qc],
                dst_ref=rs_recv.at[qc],
                send_sem=send_sems.at[qc],
                recv_sem=recv_sems.at[qc],
                device_id=(right,),
                device_id_type=MESH,
            )
            rdma.start()

        @pl.when((qc == QC - 1) & (not _SKIP_COMM))
        def _finish():
            red = lax.rem(my_pos + 1, N_DEV)
            red_rows = pl.ds(red * CH, CH)
            agb[red_rows, :] = out_ref[red_rows, :].astype(jnp.bfloat16)
            for t in range(N_DEV - 1):
                cw_c = lax.rem(my_pos + 1 - t + N_DEV, N_DEV)
                ccw_c = lax.rem(my_pos + 1 + t, N_DEV)
                cw = pltpu.make_async_remote_copy(
                    src_ref=agb.at[pl.ds(cw_c * CH, HCH), :],
                    dst_ref=agb.at[pl.ds(cw_c * CH, HCH), :],
                    send_sem=send_sems.at[3 + t],
                    recv_sem=recv_sems.at[3 + t],
                    device_id=(right,),
                    device_id_type=MESH,
                )
                ccw = pltpu.make_async_remote_copy(
                    src_ref=agb.at[pl.ds(ccw_c * CH + HCH, HCH), :],
                    dst_ref=agb.at[pl.ds(ccw_c * CH + HCH, HCH), :],
                    send_sem=send_sems.at[6 + t],
                    recv_sem=recv_sems.at[6 + t],
                    device_id=(left,),
                    device_id_type=MESH,
                )
                cw.start()
                ccw.start()
                cw.wait()
                ccw.wait()
                got_cw = pl.ds(lax.rem(my_pos - t + N_DEV, N_DEV) * CH, HCH)
                got_ccw = pl.ds(lax.rem(my_pos + 2 + t, N_DEV) * CH + HCH, HCH)
                out_ref[got_cw, :] = agb[got_cw, :].astype(jnp.float32)
                out_ref[got_ccw, :] = agb[got_ccw, :].astype(jnp.float32)
            for st in range(N_DEV - 1):
                pltpu.make_async_remote_copy(
                    src_ref=sendbuf.at[st],
                    dst_ref=rs_recv.at[st],
                    send_sem=send_sems.at[st],
                    recv_sem=recv_sems.at[st],
                    device_id=(right,),
                    device_id_type=MESH,
                ).wait_send()

    out = pl.pallas_call(
        body,
        grid=(QC,),
        in_specs=[
            pl.BlockSpec((1, SQ, DM), lambda qc: (0, 0, 0)),
            pl.BlockSpec((DM, DM), lambda qc: (0, 0)),
            pl.BlockSpec((H_PER, SKV, DH), lambda qc: (0, 0, 0)),
            pl.BlockSpec((H_PER, SKV, DH), lambda qc: (0, 0, 0)),
            pl.BlockSpec((DM, DM), lambda qc: (0, 0)),
        ],
        out_specs=pl.BlockSpec((SQ, DM), lambda qc: (0, 0)),
        out_shape=jax.ShapeDtypeStruct((SQ, DM), jnp.float32),
        scratch_shapes=[
            pltpu.VMEM((CH, SKV), jnp.float32),
            pltpu.VMEM((N_DEV - 1, CH, DM), jnp.bfloat16),
            pltpu.VMEM((N_DEV - 1, CH, DM), jnp.bfloat16),
            pltpu.VMEM((SQ, DM), jnp.bfloat16),
            pltpu.SemaphoreType.DMA((9,)),
            pltpu.SemaphoreType.DMA((9,)),
        ],
        compiler_params=pltpu.CompilerParams(collective_id=0),
    )(xb, Wqb, Kb, Vb, Wob)
    return out.reshape(1, SQ, DM)


# device time: 146184 ns/iter; 1.1857x vs baseline; 1.0296x over previous
import os

import jax
import jax.numpy as jnp
from jax import lax
from jax.experimental import pallas as pl
from jax.experimental.pallas import tpu as pltpu

_SKIP_COMM = bool(os.environ.get("SKIP_COMM"))

N_DEV = 4
SQ = 2048
SKV = 2048
H_PER = 8
DH = 128
DM = 1024
QC = 4
CH = SQ // N_DEV
HCH = CH // 2
SCALE = 0.08838834764831843
BLK = 64
MESH = pl.DeviceIdType.MESH


def kernel(x, Wq, K_ext, V_ext, Wo):
    my = lax.axis_index("i")
    xb = x.astype(jnp.bfloat16)
    Wqb = Wq.astype(jnp.bfloat16)
    Wob = Wo.astype(jnp.bfloat16)
    Kb = lax.dynamic_slice_in_dim(K_ext, my * H_PER, H_PER, axis=2)[0]
    Kb = Kb.transpose(1, 0, 2).astype(jnp.bfloat16)
    Vb = lax.dynamic_slice_in_dim(V_ext, my * H_PER, H_PER, axis=2)[0]
    Vb = Vb.transpose(1, 0, 2).astype(jnp.bfloat16)

    def body(x_ref, wq_ref, k_ref, v_ref, wo_ref, out_ref,
             bias_ref, sendbuf, rs_recv, agb, send_sems, recv_sems):
        qc = pl.program_id(0)
        my_pos = lax.axis_index("i")
        left = lax.rem(my_pos - 1 + N_DEV, N_DEV)
        right = lax.rem(my_pos + 1, N_DEV)
        chunk = lax.rem(my_pos - qc + N_DEV, N_DEV)
        rows = pl.ds(chunk * CH, CH)

        @pl.when((qc == 0) & (not _SKIP_COMM))
        def _entry_barrier():
            barrier = pltpu.get_barrier_semaphore()
            for nbr in (left, right):
                pl.semaphore_signal(barrier, inc=1, device_id=(nbr,),
                                    device_id_type=MESH)
            pl.semaphore_wait(barrier, 2)

        rowb = (lax.broadcasted_iota(jnp.int32, (CH, SKV), 0) + chunk * CH) // BLK
        colb = lax.broadcasted_iota(jnp.int32, (CH, SKV), 1) // BLK
        keep = (rowb == colb) | (colb == 0) | (lax.rem(rowb + colb, 3) == 0)
        bias_ref[...] = jnp.where(keep, 0.0, -1e9).astype(jnp.float32)

        xc = x_ref[0, rows, :]
        acc = jnp.zeros((CH, DM), jnp.float32)
        for h in range(H_PER):
            kh = k_ref[h]
            vh = v_ref[h]
            qh = jnp.dot(xc, wq_ref[:, h * DH:(h + 1) * DH],
                         preferred_element_type=jnp.float32)
            qh = (qh * SCALE).astype(jnp.bfloat16)
            sc = lax.dot_general(qh, kh, (((1,), (1,)), ((), ())),
                                 preferred_element_type=jnp.float32)
            w = jnp.exp(sc + bias_ref[...])
            denom = jnp.sum(w, axis=1, keepdims=True)
            ctx = jnp.dot(w.astype(jnp.bfloat16), vh,
                          preferred_element_type=jnp.float32) / denom
            acc = acc + jnp.dot(ctx.astype(jnp.bfloat16),
                                wo_ref[h * DH:(h + 1) * DH, :],
                                preferred_element_type=jnp.float32)
        out_ref[rows, :] = acc

        @pl.when((qc > 0) & (not _SKIP_COMM))
        def _rs_recv_add():
            st = qc - 1
            rdma = pltpu.make_async_remote_copy(
                src_ref=sendbuf.at[st],
                dst_ref=rs_recv.at[st],
                send_sem=send_sems.at[st],
                recv_sem=recv_sems.at[st],
                device_id=(left,),
                device_id_type=MESH,
            )
            rdma.wait_recv()
            out_ref[rows, :] += rs_recv[st].astype(jnp.float32)

        @pl.when((qc < QC - 1) & (not _SKIP_COMM))
        def _rs_send():
            sendbuf[qc] = out_ref[rows, :].astype(jnp.bfloat16)
            rdma = pltpu.make_async_remote_copy(
                src_ref=sendbuf.at[qc],
                dst_ref=rs_recv.at[qc],
                send_sem=send_sems.at[qc],
                recv_sem=recv_sems.at[qc],
                device_id=(right,),
                device_id_type=MESH,
            )
            rdma.start()

        @pl.when((qc == QC - 1) & (not _SKIP_COMM))
        def _finish():
            red = lax.rem(my_pos + 1, N_DEV)
            red_rows = pl.ds(red * CH, CH)
            agb[red_rows, :] = out_ref[red_rows, :].astype(jnp.bfloat16)
            for t in range(N_DEV - 1):
                cw_c = lax.rem(my_pos + 1 - t + N_DEV, N_DEV)
                ccw_c = lax.rem(my_pos + 1 + t, N_DEV)
                cw = pltpu.make_async_remote_copy(
                    src_ref=agb.at[pl.ds(cw_c * CH, HCH), :],
                    dst_ref=agb.at[pl.ds(cw_c * CH, HCH), :],
                    send_sem=send_sems.at[3 + t],
                    recv_sem=recv_sems.at[3 + t],
                    device_id=(right,),
                    device_id_type=MESH,
                )
                ccw = pltpu.make_async_remote_copy(
                    src_ref=agb.at[pl.ds(ccw_c * CH + HCH, HCH), :],
                    dst_ref=agb.at[pl.ds(ccw_c * CH + HCH, HCH), :],
                    send_sem=send_sems.at[6 + t],
                    recv_sem=recv_sems.at[6 + t],
                    device_id=(left,),
                    device_id_type=MESH,
                )
                cw.start()
                ccw.start()
                cw.wait()
                ccw.wait()
                got_cw = pl.ds(lax.rem(my_pos - t + N_DEV, N_DEV) * CH, HCH)
                got_ccw = pl.ds(lax.rem(my_pos + 2 + t, N_DEV) * CH + HCH, HCH)
                out_ref[got_cw, :] = agb[got_cw, :].astype(jnp.float32)
                out_ref[got_ccw, :] = agb[got_ccw, :].astype(jnp.float32)
            for st in range(N_DEV - 1):
                pltpu.make_async_remote_copy(
                    src_ref=sendbuf.at[st],
                    dst_ref=rs_recv.at[st],
                    send_sem=send_sems.at[st],
                    recv_sem=recv_sems.at[st],
                    device_id=(right,),
                    device_id_type=MESH,
                ).wait_send()

    out = pl.pallas_call(
        body,
        grid=(QC,),
        in_specs=[
            pl.BlockSpec((1, SQ, DM), lambda qc: (0, 0, 0)),
            pl.BlockSpec((DM, DM), lambda qc: (0, 0)),
            pl.BlockSpec((H_PER, SKV, DH), lambda qc: (0, 0, 0)),
            pl.BlockSpec((H_PER, SKV, DH), lambda qc: (0, 0, 0)),
            pl.BlockSpec((DM, DM), lambda qc: (0, 0)),
        ],
        out_specs=pl.BlockSpec((SQ, DM), lambda qc: (0, 0)),
        out_shape=jax.ShapeDtypeStruct((SQ, DM), jnp.float32),
        scratch_shapes=[
            pltpu.VMEM((CH, SKV), jnp.float32),
            pltpu.VMEM((N_DEV - 1, CH, DM), jnp.bfloat16),
            pltpu.VMEM((N_DEV - 1, CH, DM), jnp.bfloat16),
            pltpu.VMEM((SQ, DM), jnp.bfloat16),
            pltpu.SemaphoreType.DMA((9,)),
            pltpu.SemaphoreType.DMA((9,)),
        ],
        compiler_params=pltpu.CompilerParams(collective_id=0),
    )(xb, Wqb, Kb, Vb, Wob)
    return out.reshape(1, SQ, DM)


# device time: 146152 ns/iter; 1.1860x vs baseline; 1.0002x over previous
import os

import jax
import jax.numpy as jnp
from jax import lax
from jax.experimental import pallas as pl
from jax.experimental.pallas import tpu as pltpu

_SKIP_COMM = bool(os.environ.get("SKIP_COMM"))

N_DEV = 4
SQ = 2048
SKV = 2048
H_PER = 8
DH = 128
DM = 1024
QC = 4
CH = SQ // N_DEV
HCH = CH // 2
SCALE = 0.08838834764831843
BLK = 64
MESH = pl.DeviceIdType.MESH


def kernel(x, Wq, K_ext, V_ext, Wo):
    my = lax.axis_index("i")
    xb = x.astype(jnp.bfloat16)
    Wqb = Wq.astype(jnp.bfloat16)
    Wob = Wo.astype(jnp.bfloat16)
    Kb = lax.dynamic_slice_in_dim(K_ext, my * H_PER, H_PER, axis=2)[0]
    Kb = Kb.transpose(1, 0, 2).astype(jnp.bfloat16)
    Vb = lax.dynamic_slice_in_dim(V_ext, my * H_PER, H_PER, axis=2)[0]
    Vb = Vb.transpose(1, 0, 2).astype(jnp.bfloat16)

    def body(x_ref, wq_ref, k_ref, v_ref, wo_ref, out_ref,
             bias_ref, sendbuf, rs_recv, agb, send_sems, recv_sems):
        qc = pl.program_id(0)
        my_pos = lax.axis_index("i")
        left = lax.rem(my_pos - 1 + N_DEV, N_DEV)
        right = lax.rem(my_pos + 1, N_DEV)
        chunk = lax.rem(my_pos - qc + N_DEV, N_DEV)
        rows = pl.ds(chunk * CH, CH)

        @pl.when((qc == 0) & (not _SKIP_COMM))
        def _entry_barrier():
            barrier = pltpu.get_barrier_semaphore()
            for nbr in (left, right):
                pl.semaphore_signal(barrier, inc=1, device_id=(nbr,),
                                    device_id_type=MESH)
            pl.semaphore_wait(barrier, 2)

        rowb = (lax.broadcasted_iota(jnp.int32, (CH, SKV), 0) + chunk * CH) // BLK
        colb = lax.broadcasted_iota(jnp.int32, (CH, SKV), 1) // BLK
        keep = (rowb == colb) | (colb == 0) | (lax.rem(rowb + colb, 3) == 0)
        bias_ref[...] = jnp.where(keep, 0.0, -1e9).astype(jnp.float32)

        xc = x_ref[0, rows, :]
        acc = jnp.zeros((CH, DM), jnp.float32)
        for h in range(H_PER):
            kh = k_ref[h]
            vh = v_ref[h]
            qh = jnp.dot(xc, wq_ref[:, h * DH:(h + 1) * DH],
                         preferred_element_type=jnp.float32)
            qh = (qh * SCALE).astype(jnp.bfloat16)
            sc = lax.dot_general(qh, kh, (((1,), (1,)), ((), ())),
                                 preferred_element_type=jnp.float32)
            w = jnp.exp(sc + bias_ref[...])
            denom = jnp.sum(w, axis=1, keepdims=True)
            ctx = jnp.dot(w.astype(jnp.bfloat16), vh,
                          preferred_element_type=jnp.float32) / denom
            acc = acc + jnp.dot(ctx.astype(jnp.bfloat16),
                                wo_ref[h * DH:(h + 1) * DH, :],
                                preferred_element_type=jnp.float32)
        out_ref[rows, :] = acc

        @pl.when((qc > 0) & (not _SKIP_COMM))
        def _rs_recv_add():
            st = qc - 1
            rdma = pltpu.make_async_remote_copy(
                src_ref=sendbuf.at[st],
                dst_ref=rs_recv.at[st],
                send_sem=send_sems.at[st],
                recv_sem=recv_sems.at[st],
                device_id=(left,),
                device_id_type=MESH,
            )
            rdma.wait_recv()
            out_ref[rows, :] += rs_recv[st].astype(jnp.float32)

        @pl.when((qc < QC - 1) & (not _SKIP_COMM))
        def _rs_send():
            sendbuf[qc] = out_ref[rows, :].astype(jnp.bfloat16)
            rdma = pltpu.make_async_remote_copy(
                src_ref=sendbuf.at[qc],
                dst_ref=rs_recv.at[qc],
                send_sem=send_sems.at[qc],
                recv_sem=recv_sems.at[qc],
                device_id=(right,),
                device_id_type=MESH,
            )
            rdma.start()

        @pl.when((qc == QC - 1) & (not _SKIP_COMM))
        def _finish():
            red = lax.rem(my_pos + 1, N_DEV)
            red_rows = pl.ds(red * CH, CH)
            agb[red_rows, :] = out_ref[red_rows, :].astype(jnp.bfloat16)
            for t in range(N_DEV - 1):
                cw_c = lax.rem(my_pos + 1 - t + N_DEV, N_DEV)
                ccw_c = lax.rem(my_pos + 1 + t, N_DEV)
                cw = pltpu.make_async_remote_copy(
                    src_ref=agb.at[pl.ds(cw_c * CH, HCH), :],
                    dst_ref=agb.at[pl.ds(cw_c * CH, HCH), :],
                    send_sem=send_sems.at[3 + t],
                    recv_sem=recv_sems.at[3 + t],
                    device_id=(right,),
                    device_id_type=MESH,
                )
                ccw = pltpu.make_async_remote_copy(
                    src_ref=agb.at[pl.ds(ccw_c * CH + HCH, HCH), :],
                    dst_ref=agb.at[pl.ds(ccw_c * CH + HCH, HCH), :],
                    send_sem=send_sems.at[6 + t],
                    recv_sem=recv_sems.at[6 + t],
                    device_id=(left,),
                    device_id_type=MESH,
                )
                cw.start()
                ccw.start()
                cw.wait_recv()
                ccw.wait_recv()
            for t in range(N_DEV - 1):
                got_cw = pl.ds(lax.rem(my_pos - t + N_DEV, N_DEV) * CH, HCH)
                got_ccw = pl.ds(lax.rem(my_pos + 2 + t, N_DEV) * CH + HCH, HCH)
                out_ref[got_cw, :] = agb[got_cw, :].astype(jnp.float32)
                out_ref[got_ccw, :] = agb[got_ccw, :].astype(jnp.float32)
            for st in range(N_DEV - 1):
                pltpu.make_async_remote_copy(
                    src_ref=sendbuf.at[st],
                    dst_ref=rs_recv.at[st],
                    send_sem=send_sems.at[st],
                    recv_sem=recv_sems.at[st],
                    device_id=(right,),
                    device_id_type=MESH,
                ).wait_send()
            for t in range(N_DEV - 1):
                for s_idx, dev in ((3 + t, right), (6 + t, left)):
                    pltpu.make_async_remote_copy(
                        src_ref=agb.at[pl.ds(0, HCH), :],
                        dst_ref=agb.at[pl.ds(0, HCH), :],
                        send_sem=send_sems.at[s_idx],
                        recv_sem=recv_sems.at[s_idx],
                        device_id=(dev,),
                        device_id_type=MESH,
                    ).wait_send()

    out = pl.pallas_call(
        body,
        grid=(QC,),
        in_specs=[
            pl.BlockSpec((1, SQ, DM), lambda qc: (0, 0, 0)),
            pl.BlockSpec((DM, DM), lambda qc: (0, 0)),
            pl.BlockSpec((H_PER, SKV, DH), lambda qc: (0, 0, 0)),
            pl.BlockSpec((H_PER, SKV, DH), lambda qc: (0, 0, 0)),
            pl.BlockSpec((DM, DM), lambda qc: (0, 0)),
        ],
        out_specs=pl.BlockSpec((SQ, DM), lambda qc: (0, 0)),
        out_shape=jax.ShapeDtypeStruct((SQ, DM), jnp.float32),
        scratch_shapes=[
            pltpu.VMEM((CH, SKV), jnp.float32),
            pltpu.VMEM((N_DEV - 1, CH, DM), jnp.bfloat16),
            pltpu.VMEM((N_DEV - 1, CH, DM), jnp.bfloat16),
            pltpu.VMEM((SQ, DM), jnp.bfloat16),
            pltpu.SemaphoreType.DMA((9,)),
            pltpu.SemaphoreType.DMA((9,)),
        ],
        compiler_params=pltpu.CompilerParams(collective_id=0),
    )(xb, Wqb, Kb, Vb, Wob)
    return out.reshape(1, SQ, DM)
